# baseline (speedup 1.0000x reference)
import functools

import jax
import jax.numpy as jnp
import numpy as np
from jax import lax
from jax.experimental import pallas as pl
from jax.experimental.pallas import tpu as pltpu
from jax.experimental.pallas import tpu_sc as plsc

_S = 512
_D = 64
_B = 2
_OFF = 2
_NTRI = (_S - _OFF) * (_S - _OFF + 1) // 2

_CHUNK = 128
_NW = 32
_CH_PER_W = 64
_KPAD = 130312
_NFULL = _NTRI // _CHUNK
_CH_PER_B = _NFULL + 1
_NCH_VALID = _B * _CH_PER_B
_LAST_BASE = _KPAD - _CHUNK
_NBUF = 4


def _build_index_chunks() -> np.ndarray:
    r, c = np.triu_indices(_S, _OFF)
    idx0 = (r + _S * c).astype(np.int32)
    chunks = np.zeros((_NW * _CH_PER_W, _CHUNK), np.int32)
    for b in range(_B):
        per_b = idx0 + b * _S * _S
        full = per_b[: _NFULL * _CHUNK].reshape(_NFULL, _CHUNK)
        chunks[b * _CH_PER_B:b * _CH_PER_B + _NFULL] = full
        tail = per_b[_LAST_BASE:]
        chunks[b * _CH_PER_B + _NFULL, :len(tail)] = tail
    return chunks.reshape(_NW, _CH_PER_W, _CHUNK)


_IDX_CHUNKS = _build_index_chunks()


@functools.cache
def _make_triu_gather():
    mesh = plsc.VectorSubcoreMesh(
        core_axis_name="c", subcore_axis_name="s", num_cores=2, num_subcores=16
    )
    return functools.partial(
        pl.kernel,
        out_type=jax.ShapeDtypeStruct((_B, _D, _KPAD), jnp.float32),
        mesh=mesh,
        compiler_params=pltpu.CompilerParams(
            use_tc_tiling_on_sc=False, needs_layout_passes=False
        ),
        scratch_types=[
            pltpu.VMEM((_CH_PER_W, _CHUNK), jnp.int32),
            [pltpu.VMEM((_CHUNK, _D), jnp.float32)] * _NBUF,
            [pltpu.VMEM((_D, _CHUNK), jnp.float32)] * _NBUF,
            [pltpu.SemaphoreType.DMA] * _NBUF,
            [pltpu.SemaphoreType.DMA] * _NBUF,
        ],
    )(_triu_gather)


def _triu_gather(table_hbm, idx_hbm, out_hbm, idx_v, gbufs, tbufs,
                 gsems, wsems):
    wid = lax.axis_index("s") * 2 + lax.axis_index("c")
    c0 = wid * _CH_PER_W
    pltpu.sync_copy(idx_hbm.at[wid], idx_v)

    def chunk_ok(j):
        return jnp.logical_and(j < _CH_PER_W, c0 + j < _NCH_VALID)

    def dst(j):
        c = c0 + j
        b = (c >= _CH_PER_B).astype(jnp.int32)
        pos = c - b * _CH_PER_B
        base = pl.multiple_of(lax.min(pos * _CHUNK, _LAST_BASE), 8)
        return out_hbm.at[b, :, pl.ds(base, _CHUNK)]

    def gather_start(j, s):
        @pl.when(chunk_ok(j))
        def _():
            pltpu.async_copy(table_hbm.at[idx_v.at[j]], gbufs[s], gsems[s])

    def gather_wait(j, s):
        @pl.when(chunk_ok(j))
        def _():
            pltpu.make_async_copy(table_hbm.at[idx_v.at[j]], gbufs[s],
                                  gsems[s]).wait()

    k16 = lax.iota(jnp.int32, 16)
    kidx = [k16 + (kg * 16) for kg in range(_CHUNK // 16)]

    def transpose(j, s):
        @pl.when(chunk_ok(j))
        def _():
            gb, tb = gbufs[s], tbufs[s]

            def body(i, carry):
                d0 = i * 8
                for dd in range(8):
                    d = d0 + dd
                    d16 = jnp.broadcast_to(d, (16,))
                    for kg in range(_CHUNK // 16):
                        v = plsc.load_gather(gb, [kidx[kg], d16])
                        tb[d, pl.ds(kg * 16, 16)] = v
                return carry

            lax.fori_loop(0, _D // 8, body, 0)

    def write_start(j, s):
        @pl.when(chunk_ok(j))
        def _():
            pltpu.async_copy(tbufs[s], dst(j), wsems[s])

    def write_wait(j, s, extra_ok=None):
        ok = chunk_ok(j) if extra_ok is None else jnp.logical_and(
            chunk_ok(j), extra_ok)

        @pl.when(ok)
        def _():
            pltpu.make_async_copy(tbufs[s], dst(j), wsems[s]).wait()

    for s in range(_NBUF):
        gather_start(s, s)

    def step(i, carry):
        j0 = i * _NBUF
        for s in range(_NBUF):
            j = j0 + s
            write_wait(j - _NBUF, s, extra_ok=j - _NBUF >= 0)
            gather_wait(j, s)
            transpose(j, s)
            gather_start(j + _NBUF, s)
            write_start(j, s)
        return carry

    lax.fori_loop(0, _CH_PER_W // _NBUF, step, 0)
    for s in range(_NBUF):
        write_wait(_CH_PER_W - _NBUF + s, s)


def kernel(inputs):
    table = inputs.reshape(_B * _S * _S, _D)
    out = _make_triu_gather()(table, jnp.asarray(_IDX_CHUNKS))
    return jnp.swapaxes(out[:, :, :_NTRI], 1, 2)


# speedup vs baseline: 1.7165x; 1.7165x over previous
import functools

import jax
import jax.numpy as jnp
import numpy as np
from jax import lax
from jax.experimental import pallas as pl
from jax.experimental.pallas import tpu as pltpu
from jax.experimental.pallas import tpu_sc as plsc

_S = 512
_D = 64
_B = 2
_OFF = 2
_NTRI = (_S - _OFF) * (_S - _OFF + 1) // 2
_TROWS = _B * _S * _S // 2

_CHUNK = 128
_NW = 32
_CH_PER_W = 64
_CH_PER_B = _NTRI // _CHUNK
_NCH_VALID = _B * _CH_PER_B
_TAIL_K = _CH_PER_B * _CHUNK
_TAIL_SLOT = _NCH_VALID - (_NW - 1) * _CH_PER_W
_NBUF = 3


def _build_tables() -> tuple[np.ndarray, np.ndarray]:
    r, c = np.triu_indices(_S, _OFF)
    f = (r + _S * c).astype(np.int64)
    pairs = np.zeros((_NW * _CH_PER_W, _CHUNK), np.int32)
    halfs = np.zeros((_NW * _CH_PER_W, _CHUNK), np.int32)
    for b in range(_B):
        fb = f + b * _S * _S
        p = (fb // 2).astype(np.int32)
        h = ((fb % 2) * _D).astype(np.int32)
        full = slice(0, _CH_PER_B * _CHUNK)
        rows = slice(b * _CH_PER_B, (b + 1) * _CH_PER_B)
        pairs[rows] = p[full].reshape(_CH_PER_B, _CHUNK)
        halfs[rows] = h[full].reshape(_CH_PER_B, _CHUNK)
    return (pairs.reshape(_NW, _CH_PER_W, _CHUNK),
            halfs.reshape(_NW, _CH_PER_W, _CHUNK))


_PAIRS, _HALFS = _build_tables()


@functools.cache
def _make_triu_gather():
    mesh = plsc.VectorSubcoreMesh(
        core_axis_name="c", subcore_axis_name="s", num_cores=2, num_subcores=16
    )
    return functools.partial(
        pl.kernel,
        out_type=jax.ShapeDtypeStruct((_B, _D, _NTRI), jnp.float32),
        mesh=mesh,
        compiler_params=pltpu.CompilerParams(needs_layout_passes=False),
        scratch_types=[
            pltpu.VMEM((_CH_PER_W, _CHUNK), jnp.int32),
            pltpu.VMEM((_CH_PER_W, _CHUNK), jnp.int32),
            [pltpu.VMEM((_CHUNK, 2 * _D), jnp.float32)] * _NBUF,
            [pltpu.VMEM((_D, _CHUNK), jnp.float32)] * _NBUF,
            [pltpu.SemaphoreType.DMA] * _NBUF,
            [pltpu.SemaphoreType.DMA] * _NBUF,
        ],
    )(_triu_gather)


def _triu_gather(table_hbm, pair_hbm, half_hbm, out_hbm, pair_v, half_v,
                 gbufs, tbufs, gsems, wsems):
    wid = lax.axis_index("s") * 2 + lax.axis_index("c")
    c0 = wid * _CH_PER_W
    pltpu.sync_copy(pair_hbm.at[wid], pair_v)
    pltpu.sync_copy(half_hbm.at[wid], half_v)

    def chunk_ok(j):
        return jnp.logical_and(j < _CH_PER_W, c0 + j < _NCH_VALID)

    def dst(j):
        c = c0 + j
        b = (c >= _CH_PER_B).astype(jnp.int32)
        pos = c - b * _CH_PER_B
        base = pl.multiple_of(pos * _CHUNK, _CHUNK)
        return out_hbm.at[b, :, pl.ds(base, _CHUNK)]

    def gather_start(j, s):
        @pl.when(chunk_ok(j))
        def _():
            pltpu.async_copy(table_hbm.at[pair_v.at[j]], gbufs[s], gsems[s])

    def gather_wait(j, s):
        @pl.when(chunk_ok(j))
        def _():
            pltpu.make_async_copy(table_hbm.at[pair_v.at[j]], gbufs[s],
                                  gsems[s]).wait()

    k16 = lax.iota(jnp.int32, 16)
    kidx = [k16 + (kg * 16) for kg in range(_CHUNK // 16)]

    def transpose(j, s):
        @pl.when(chunk_ok(j))
        def _():
            gb, tb = gbufs[s], tbufs[s]
            hvec = [half_v[j, pl.ds(kg * 16, 16)]
                    for kg in range(_CHUNK // 16)]

            def body(i, carry):
                d0 = i * 8
                for dd in range(8):
                    d = d0 + dd
                    d16 = jnp.broadcast_to(d, (16,))
                    for kg in range(_CHUNK // 16):
                        col = d16 + hvec[kg]
                        v = plsc.load_gather(gb, [kidx[kg], col])
                        tb[d, pl.ds(kg * 16, 16)] = v
                return carry

            lax.fori_loop(0, _D // 8, body, 0)

    def write_start(j, s):
        @pl.when(chunk_ok(j))
        def _():
            pltpu.async_copy(tbufs[s], dst(j), wsems[s])

    def write_wait(j, s, extra_ok=None):
        ok = chunk_ok(j) if extra_ok is None else jnp.logical_and(
            chunk_ok(j), extra_ok)

        @pl.when(ok)
        def _():
            pltpu.make_async_copy(tbufs[s], dst(j), wsems[s]).wait()

    for s in range(_NBUF):
        gather_start(s, s)

    def step(i, carry):
        j0 = i * _NBUF
        for s in range(_NBUF):
            j = j0 + s
            write_wait(j - _NBUF, s, extra_ok=j - _NBUF >= 0)
            gather_wait(j, s)
            transpose(j, s)
            gather_start(j + _NBUF, s)
            write_start(j, s)
        return carry

    n_steps = -(-_CH_PER_W // _NBUF)
    lax.fori_loop(0, n_steps, step, 0)
    for s in range(_NBUF):
        write_wait((n_steps - 1) * _NBUF + s, s)


def kernel(inputs):
    table = inputs.reshape(_TROWS, 2 * _D)
    out = _make_triu_gather()(table, jnp.asarray(_PAIRS), jnp.asarray(_HALFS))
    out = jnp.swapaxes(out, 1, 2)
    tail = inputs[:, _S - 1, _S - _OFF - 1, :][:, None, :]
    return lax.dynamic_update_slice(out, tail, (0, _TAIL_K, 0))


# speedup vs baseline: 2.3991x; 1.3977x over previous
import functools

import jax
import jax.numpy as jnp
import numpy as np
from jax import lax
from jax.experimental import pallas as pl
from jax.experimental.pallas import tpu as pltpu
from jax.experimental.pallas import tpu_sc as plsc

_S = 512
_D = 64
_B = 2
_OFF = 2
_NTRI = (_S - _OFF) * (_S - _OFF + 1) // 2
_TROWS = _B * _S * _S // 2

_CHUNK = 128
_NW = 32
_CH_PER_W = 64
_CH_PER_B = _NTRI // _CHUNK
_NCH_VALID = _B * _CH_PER_B
_TAIL_K = _CH_PER_B * _CHUNK
_TAIL_SLOT = _NCH_VALID - (_NW - 1) * _CH_PER_W
_NBUF = 3


def _build_tables() -> tuple[np.ndarray, np.ndarray]:
    r, c = np.triu_indices(_S, _OFF)
    f = (r + _S * c).astype(np.int64)
    pairs = np.zeros((_NW * _CH_PER_W, _CHUNK), np.int32)
    halfs = np.zeros((_NW * _CH_PER_W, _CHUNK), np.int32)
    for b in range(_B):
        fb = f + b * _S * _S
        p = (fb // 2).astype(np.int32)
        h = ((fb % 2) * _D).astype(np.int32)
        full = slice(0, _CH_PER_B * _CHUNK)
        rows = slice(b * _CH_PER_B, (b + 1) * _CH_PER_B)
        pairs[rows] = p[full].reshape(_CH_PER_B, _CHUNK)
        halfs[rows] = h[full].reshape(_CH_PER_B, _CHUNK)
    return (pairs.reshape(_NW, _CH_PER_W, _CHUNK),
            halfs.reshape(_NW, _CH_PER_W, _CHUNK))


_PAIRS, _HALFS = _build_tables()


@functools.cache
def _make_triu_gather():
    mesh = plsc.VectorSubcoreMesh(
        core_axis_name="c", subcore_axis_name="s", num_cores=2, num_subcores=16
    )
    return functools.partial(
        pl.kernel,
        out_type=jax.ShapeDtypeStruct((_B, _D, _NTRI), jnp.float32),
        mesh=mesh,
        compiler_params=pltpu.CompilerParams(needs_layout_passes=False),
        scratch_types=[
            pltpu.VMEM((_CH_PER_W, _CHUNK), jnp.int32),
            pltpu.VMEM((_CH_PER_W, _CHUNK), jnp.int32),
            [pltpu.VMEM((_CHUNK, 2 * _D), jnp.float32)] * _NBUF,
            [pltpu.VMEM((_D, _CHUNK), jnp.float32)] * _NBUF,
            [pltpu.SemaphoreType.DMA] * _NBUF,
            [pltpu.SemaphoreType.DMA] * _NBUF,
        ],
    )(_triu_gather)


def _triu_gather(table_hbm, pair_hbm, half_hbm, out_hbm, pair_v, half_v,
                 gbufs, tbufs, gsems, wsems):
    wid = lax.axis_index("s") * 2 + lax.axis_index("c")
    c0 = wid * _CH_PER_W
    pltpu.sync_copy(pair_hbm.at[wid], pair_v)
    pltpu.sync_copy(half_hbm.at[wid], half_v)

    def chunk_ok(j):
        return jnp.logical_and(j < _CH_PER_W, c0 + j < _NCH_VALID)

    def dst(j):
        c = c0 + j
        b = (c >= _CH_PER_B).astype(jnp.int32)
        pos = c - b * _CH_PER_B
        base = pl.multiple_of(pos * _CHUNK, _CHUNK)
        return out_hbm.at[b, :, pl.ds(base, _CHUNK)]

    def gather_start(j, s):
        @pl.when(chunk_ok(j))
        def _():
            pltpu.async_copy(table_hbm.at[pair_v.at[j]], gbufs[s], gsems[s])

    def gather_wait(j, s):
        @pl.when(chunk_ok(j))
        def _():
            pltpu.make_async_copy(table_hbm.at[pair_v.at[j]], gbufs[s],
                                  gsems[s]).wait()

    k16 = lax.iota(jnp.int32, 16)
    kidx = [k16 + (kg * 16) for kg in range(_CHUNK // 16)]
    rot = [jnp.bitwise_and(k16 + rr, 15) for rr in range(16)]

    def transpose(j, s):
        @pl.when(chunk_ok(j))
        def _():
            gb, tb = gbufs[s], tbufs[s]
            hvec = [half_v[j, pl.ds(kg * 16, 16)]
                    for kg in range(_CHUNK // 16)]

            def body(i, carry):
                d0 = i * 16
                for kg in range(_CHUNK // 16):
                    base = hvec[kg] + d0
                    for rr in range(16):
                        col = base + rot[rr]
                        v = plsc.load_gather(gb, [kidx[kg], col])
                        plsc.store_scatter(tb, [rot[rr] + d0, kidx[kg]], v)
                return carry

            lax.fori_loop(0, _D // 16, body, 0)

    def write_start(j, s):
        @pl.when(chunk_ok(j))
        def _():
            pltpu.async_copy(tbufs[s], dst(j), wsems[s])

    def write_wait(j, s, extra_ok=None):
        ok = chunk_ok(j) if extra_ok is None else jnp.logical_and(
            chunk_ok(j), extra_ok)

        @pl.when(ok)
        def _():
            pltpu.make_async_copy(tbufs[s], dst(j), wsems[s]).wait()

    for s in range(_NBUF):
        gather_start(s, s)

    def step(i, carry):
        j0 = i * _NBUF
        for s in range(_NBUF):
            j = j0 + s
            write_wait(j - _NBUF, s, extra_ok=j - _NBUF >= 0)
            gather_wait(j, s)
            transpose(j, s)
            gather_start(j + _NBUF, s)
            write_start(j, s)
        return carry

    n_steps = -(-_CH_PER_W // _NBUF)
    lax.fori_loop(0, n_steps, step, 0)
    for s in range(_NBUF):
        write_wait((n_steps - 1) * _NBUF + s, s)


def kernel(inputs):
    table = inputs.reshape(_TROWS, 2 * _D)
    out = _make_triu_gather()(table, jnp.asarray(_PAIRS), jnp.asarray(_HALFS))
    out = jnp.swapaxes(out, 1, 2)
    tail = inputs[:, _S - 1, _S - _OFF - 1, :][:, None, :]
    return lax.dynamic_update_slice(out, tail, (0, _TAIL_K, 0))
